# baseline (speedup 1.0000x reference)
import functools

import jax
import jax.numpy as jnp
from jax import lax
from jax.experimental import pallas as pl
from jax.experimental.pallas import tpu as pltpu
from jax.experimental.pallas import tpu_sc as plsc

F = 26
B = 16384
V = 100000
D = 16

NC = 2
NS = 16
NW = NC * NS
L = 16

ROWS = F * B
BPW = B // NW
RPW = BPW * F
CHUNK = 512
NCH = RPW // CHUNK
NPAIR = NCH // 2


def _body(tab, idx, out, ibuf, fbuf, buf0, buf1, gsem0, gsem1, wsem0, wsem1):
    bufs = (buf0, buf1)
    gsems = (gsem0, gsem1)
    wsems = (wsem0, wsem1)

    wid = lax.axis_index("s") * NC + lax.axis_index("c")
    b0 = wid * BPW
    base = wid * RPW

    pltpu.sync_copy(idx.at[:, pl.ds(b0, BPW)], ibuf)

    iota = lax.iota(jnp.int32, L)

    def build_f(f, c):
        off = f * V
        for cb in range(BPW // L):
            v = ibuf[f, pl.ds(cb * L, L)] + off
            tgt = (iota + cb * L) * F + f
            plsc.store_scatter(fbuf, [tgt], v)
        return c

    lax.fori_loop(0, F, build_f, 0)

    def gather_dma(g, p):
        iref = fbuf.at[pl.ds(g * CHUNK, CHUNK)]
        return pltpu.make_async_copy(tab.at[iref], bufs[p], gsems[p])

    def write_dma(g, p):
        return pltpu.make_async_copy(
            bufs[p], out.at[pl.ds(base + g * CHUNK, CHUNK)], wsems[p]
        )

    gather_dma(0, 0).start()
    gather_dma(1, 1).start()

    def pair_body(h, c):
        g0 = 2 * h
        gather_dma(g0, 0).wait()
        write_dma(g0, 0).start()
        gather_dma(g0 + 1, 1).wait()
        write_dma(g0 + 1, 1).start()
        write_dma(g0, 0).wait()
        gather_dma(g0 + 2, 0).start()
        write_dma(g0 + 1, 1).wait()
        gather_dma(g0 + 3, 1).start()
        return c

    lax.fori_loop(0, NPAIR - 1, pair_body, 0)

    gl = NCH - 2
    gather_dma(gl, 0).wait()
    write_dma(gl, 0).start()
    gather_dma(gl + 1, 1).wait()
    write_dma(gl + 1, 1).start()
    write_dma(gl, 0).wait()
    write_dma(gl + 1, 1).wait()


def kernel(tables, indices):
    mesh = plsc.VectorSubcoreMesh(core_axis_name="c", subcore_axis_name="s")
    run = functools.partial(
        pl.kernel,
        mesh=mesh,
        compiler_params=pltpu.CompilerParams(use_tc_tiling_on_sc=False, needs_layout_passes=False),
        out_type=jax.ShapeDtypeStruct((ROWS, D), jnp.float32),
        scratch_types=[
            pltpu.VMEM((F, BPW), jnp.int32),
            pltpu.VMEM((RPW,), jnp.int32),
            pltpu.VMEM((CHUNK, D), jnp.float32),
            pltpu.VMEM((CHUNK, D), jnp.float32),
            pltpu.SemaphoreType.DMA,
            pltpu.SemaphoreType.DMA,
            pltpu.SemaphoreType.DMA,
            pltpu.SemaphoreType.DMA,
        ],
    )(_body)
    out = run(tables.reshape(F * V, D), indices)
    return out.reshape(B, F, D)


# speedup vs baseline: 7.0763x; 7.0763x over previous
import functools

import jax
import jax.numpy as jnp
from jax import lax
from jax.experimental import pallas as pl
from jax.experimental.pallas import tpu as pltpu
from jax.experimental.pallas import tpu_sc as plsc

F = 26
B = 16384
V = 100000
D = 16
L = 16

NC = 2
NS = 16
NW = NC * NS

PLANES = F * D
PPW = PLANES // NW
OCH = 4096
NOCH = B // OCH


def _body(tab, idx, out, ibuf, pbuf, obuf):
    wid = lax.axis_index("s") * NC + lax.axis_index("c")
    p0 = wid * PPW

    def do_plane(j):
        p = p0 + j
        f = p // D
        d = p % D

        @pl.when(jnp.logical_or(j == 0, d == 0))
        def _():
            pltpu.sync_copy(idx.at[f], ibuf)

        pltpu.sync_copy(tab.at[f, d], pbuf)

        for oc in range(NOCH):
            def it(i, c):
                iv = ibuf[pl.ds(oc * OCH + i * L, L)]
                obuf[pl.ds(i * L, L)] = plsc.load_gather(pbuf, [iv])
                return c

            lax.fori_loop(0, OCH // L, it, 0, unroll=8)
            pltpu.sync_copy(obuf, out.at[f, d, pl.ds(oc * OCH, OCH)])

    for j in range(PPW):
        do_plane(j)


def kernel(tables, indices):
    mesh = plsc.VectorSubcoreMesh(core_axis_name="c", subcore_axis_name="s")
    run = functools.partial(
        pl.kernel,
        mesh=mesh,
        compiler_params=pltpu.CompilerParams(
            use_tc_tiling_on_sc=True, needs_layout_passes=False
        ),
        out_type=jax.ShapeDtypeStruct((F, D, B), jnp.float32),
        scratch_types=[
            pltpu.VMEM((B,), jnp.int32),
            pltpu.VMEM((V,), jnp.float32),
            pltpu.VMEM((OCH,), jnp.float32),
        ],
    )(_body)
    out_t = run(tables.transpose(0, 2, 1), indices)
    return jnp.transpose(out_t, (2, 0, 1))


# speedup vs baseline: 7.3167x; 1.0340x over previous
import functools

import jax
import jax.numpy as jnp
from jax import lax
from jax.experimental import pallas as pl
from jax.experimental.pallas import tpu as pltpu
from jax.experimental.pallas import tpu_sc as plsc

F = 26
B = 16384
V = 100000
D = 16
L = 16

NC = 2
NS = 16
NW = NC * NS

PLANES = F * D
PPW = PLANES // NW
OCH = 4096
NOCH = B // OCH


def _body(tab, idx, out, ibuf, pbuf, ob0, ob1, isem, wsem0, wsem1):
    obufs = (ob0, ob1)
    wsems = (wsem0, wsem1)

    wid = lax.axis_index("s") * NC + lax.axis_index("c")
    p0 = wid * PPW

    g = 0
    for j in range(PPW):
        p = p0 + j
        f = p // D
        d = p % D
        restage = jnp.logical_or(j == 0, d == 0)

        @pl.when(restage)
        def _():
            pltpu.make_async_copy(idx.at[f], ibuf, isem).start()

        pltpu.sync_copy(tab.at[f, d], pbuf)

        @pl.when(restage)
        def _():
            pltpu.make_async_copy(idx.at[f], ibuf, isem).wait()

        for oc in range(NOCH):
            par = g % 2
            bb = obufs[par]
            dst = out.at[f, d, pl.ds(oc * OCH, OCH)]
            if g >= 2:
                pltpu.make_async_copy(bb, dst, wsems[par]).wait()

            def it(i, c):
                iv = ibuf[pl.ds(oc * OCH + i * L, L)]
                bb[pl.ds(i * L, L)] = plsc.load_gather(pbuf, [iv])
                return c

            lax.fori_loop(0, OCH // L, it, 0, unroll=16)
            pltpu.make_async_copy(bb, dst, wsems[par]).start()
            g += 1

    for par in range(2):
        pltpu.make_async_copy(
            obufs[par], out.at[0, 0, pl.ds(0, OCH)], wsems[par]
        ).wait()


def kernel(tables, indices):
    mesh = plsc.VectorSubcoreMesh(core_axis_name="c", subcore_axis_name="s")
    run = functools.partial(
        pl.kernel,
        mesh=mesh,
        compiler_params=pltpu.CompilerParams(
            use_tc_tiling_on_sc=True, needs_layout_passes=False
        ),
        out_type=jax.ShapeDtypeStruct((F, D, B), jnp.float32),
        scratch_types=[
            pltpu.VMEM((B,), jnp.int32),
            pltpu.VMEM((V,), jnp.float32),
            pltpu.VMEM((OCH,), jnp.float32),
            pltpu.VMEM((OCH,), jnp.float32),
            pltpu.SemaphoreType.DMA,
            pltpu.SemaphoreType.DMA,
            pltpu.SemaphoreType.DMA,
        ],
    )(_body)
    out_t = run(tables.transpose(0, 2, 1), indices)
    return jnp.transpose(out_t, (2, 0, 1))


# speedup vs baseline: 13.7863x; 1.8842x over previous
import functools

import jax
import jax.numpy as jnp
from jax import lax
from jax.experimental import pallas as pl
from jax.experimental.pallas import tpu as pltpu
from jax.experimental.pallas import tpu_sc as plsc

F = 26
B = 16384
V = 100000
D = 16
L = 16

NC = 2
NS = 16
NW = NC * NS

PLANES = F * D
PPW = PLANES // NW
OCH = 4096
NOCH = B // OCH


def _body(tab, idx, out, ibuf, pbuf, ob0, ob1, isem, wsem0, wsem1):
    obufs = (ob0, ob1)
    wsems = (wsem0, wsem1)

    wid = lax.axis_index("s") * NC + lax.axis_index("c")
    p0 = wid * PPW

    g = 0
    for j in range(PPW):
        p = p0 + j
        f = p // D
        d = p % D
        restage = jnp.logical_or(j == 0, d == 0)

        @pl.when(restage)
        def _():
            pltpu.make_async_copy(idx.at[f], ibuf, isem).start()

        pltpu.sync_copy(tab.at[f, d], pbuf)

        @pl.when(restage)
        def _():
            pltpu.make_async_copy(idx.at[f], ibuf, isem).wait()

        for oc in range(NOCH):
            par = g % 2
            bb = obufs[par]
            dst = out.at[f, d, pl.ds(oc * OCH, OCH)]
            if g >= 2:
                pltpu.make_async_copy(bb, dst, wsems[par]).wait()

            @plsc.parallel_loop(0, OCH, step=L, unroll=8)
            def _(i):
                iv = ibuf[pl.ds(oc * OCH + i, L)]
                bb[pl.ds(i, L)] = plsc.load_gather(pbuf, [iv])
            pltpu.make_async_copy(bb, dst, wsems[par]).start()
            g += 1

    for par in range(2):
        pltpu.make_async_copy(
            obufs[par], out.at[0, 0, pl.ds(0, OCH)], wsems[par]
        ).wait()


def kernel(tables, indices):
    mesh = plsc.VectorSubcoreMesh(core_axis_name="c", subcore_axis_name="s")
    run = functools.partial(
        pl.kernel,
        mesh=mesh,
        compiler_params=pltpu.CompilerParams(
            use_tc_tiling_on_sc=True, needs_layout_passes=False
        ),
        out_type=jax.ShapeDtypeStruct((F, D, B), jnp.float32),
        scratch_types=[
            pltpu.VMEM((B,), jnp.int32),
            pltpu.VMEM((V,), jnp.float32),
            pltpu.VMEM((OCH,), jnp.float32),
            pltpu.VMEM((OCH,), jnp.float32),
            pltpu.SemaphoreType.DMA,
            pltpu.SemaphoreType.DMA,
            pltpu.SemaphoreType.DMA,
        ],
    )(_body)
    out_t = run(tables.transpose(0, 2, 1), indices)
    return jnp.transpose(out_t, (2, 0, 1))
